# baseline (speedup 1.0000x reference)
import functools

import jax
import jax.numpy as jnp
from jax import lax
from jax.experimental import pallas as pl
from jax.experimental.pallas import tpu as pltpu
from jax.experimental.pallas import tpu_sc as plsc


def _first_hit_indices(pp, vertex, vn2, t2, B, N, NV, KP):
    NB = 256
    NG = N // NB

    vert_p = jnp.concatenate(
        [vertex, jnp.zeros((NV, KP - 3), jnp.float32)], axis=1)
    vv = jnp.stack([vn2, jnp.arange(NV, dtype=jnp.float32)], axis=1)
    t2s = t2.reshape(1, 1)

    def body(t2_ref, pp_ref, vert_ref, vv_ref, idx_ref):
        cross = lax.dot_general(
            vert_ref[...], pp_ref[0],
            (((1,), (0,)), ((), ())),
            preferred_element_type=jnp.float32)
        d2 = (pp_ref[0, 3:4, :] + vv_ref[:, 0:1]) - 2.0 * cross
        mask = d2 <= t2_ref[0, 0]
        cand = jnp.where(mask, vv_ref[:, 1:2], float(NV))
        first = jnp.min(cand, axis=0, keepdims=True)
        idx_ref[0] = first.astype(jnp.int32)

    idx3 = pl.pallas_call(
        body,
        grid=(B, NG),
        in_specs=[
            pl.BlockSpec(memory_space=pltpu.SMEM),
            pl.BlockSpec((1, KP, NB), lambda b, g: (b, 0, g)),
            pl.BlockSpec((NV, KP), lambda b, g: (0, 0)),
            pl.BlockSpec((NV, 2), lambda b, g: (0, 0)),
        ],
        out_specs=pl.BlockSpec((1, 1, NB), lambda b, g: (b * NG + g, 0, 0)),
        out_shape=jax.ShapeDtypeStruct((B * NG, 1, NB), jnp.int32),
    )(t2s, pp, vert_p, vv)
    return idx3.reshape(B * N)


def _sc_gather(table, idx_local, B, N, NVp, D):
    info = plsc.get_sparse_core_info()
    NC = info.num_cores
    NS = info.num_subcores
    PPT = N // NS
    RPT = NVp // NS
    mesh = plsc.VectorSubcoreMesh(core_axis_name="c", subcore_axis_name="s")

    @functools.partial(
        pl.kernel, mesh=mesh,
        out_type=jax.ShapeDtypeStruct((B * N, D), jnp.float32),
        scratch_types=[
            pltpu.VMEM((PPT,), jnp.int32),
            pltpu.VMEM((PPT, D), jnp.float32),
            pltpu.VMEM_SHARED((NVp, D), jnp.float32),
            pltpu.SemaphoreType.DMA,
        ],
    )
    def gather_k(tbl_hbm, idx_hbm, out_hbm, idx_v, rows_v, shared, sem):
        c = lax.axis_index("c")
        s = lax.axis_index("s")
        pltpu.sync_copy(tbl_hbm.at[pl.ds(c * NVp + s * RPT, RPT)],
                        shared.at[pl.ds(s * RPT, RPT)])
        base = c * N + s * PPT
        pltpu.sync_copy(idx_hbm.at[pl.ds(base, PPT)], idx_v)
        plsc.subcore_barrier()
        pltpu.async_copy(shared.at[idx_v], rows_v, sem).wait()
        pltpu.sync_copy(rows_v, out_hbm.at[pl.ds(base, PPT)])

    return gather_k(table, idx_local)


def _mlp(original, feat, W1, b1, g1, be1, W2, b2, g2, be2, B, C19, C0):
    Ltot = float(B * (C19 + C0))
    col = lambda v: v.reshape(-1, 1)
    eps = 1e-5

    def body(orig_ref, feat_ref, W1_ref, W2_ref, b1_ref, g1_ref,
             be1_ref, b2_ref, g2_ref, be2_ref, out_ref):
        W1_ = W1_ref[...]
        cdims = (((1,), (0,)), ((), ()))
        hA = [lax.dot_general(W1_, orig_ref[b], (((1,), (1,)), ((), ())),
                              preferred_element_type=jnp.float32)
              + b1_ref[...] for b in range(B)]
        hF = [lax.dot_general(W1_, feat_ref[b][:, :C0], cdims,
                              preferred_element_type=jnp.float32)
              + b1_ref[...] for b in range(B)]

        def bn_relu(parts, g_ref, be_ref):
            s = sum(jnp.sum(h, axis=1, keepdims=True) for h in parts)
            mean = s / Ltot
            q = sum(jnp.sum((h - mean) ** 2, axis=1, keepdims=True)
                    for h in parts)
            scale = g_ref[...] / jnp.sqrt(q / Ltot + eps)
            return [jnp.maximum((h - mean) * scale + be_ref[...], 0.0)
                    for h in parts]

        y1 = bn_relu(hA + hF, g1_ref, be1_ref)
        W2_ = W2_ref[...]
        z = [lax.dot_general(W2_, y, cdims,
                             preferred_element_type=jnp.float32)
             + b2_ref[...] for y in y1]
        y2 = bn_relu(z, g2_ref, be2_ref)
        for b in range(B):
            out_ref[b, :, :C19] = y2[b]
            out_ref[b, :, C19:] = y2[B + b]

    M = W1.shape[0]
    return pl.pallas_call(
        body,
        out_shape=jax.ShapeDtypeStruct((B, M, C19 + C0), jnp.float32),
    )(original, feat, W1, W2, col(b1), col(g1), col(be1),
      col(b2), col(g2), col(be2))


def kernel(original, projected, vertex, near_idx,
           W1, b1, g1, be1, W2, b2, g2, be2):
    B, C19, N = original.shape
    NV = vertex.shape[0]
    C0 = projected.shape[1]

    r = jnp.sqrt(jnp.sum(vertex[0] ** 2))
    vn2 = jnp.sum(vertex ** 2, axis=-1)
    xyzr = original[:, :3, :].reshape(B * 3, N // 128, 128)
    x, y, z = xyzr[0::3], xyzr[1::3], xyzr[2::3]
    r_pt = jnp.sqrt(jnp.sum(jnp.stack([x, y, z], axis=-1) ** 2, axis=-1))
    theta = jnp.arccos(jnp.clip(z / jnp.maximum(r_pt, 1e-12), -1.0, 1.0))
    phi = jnp.arctan2(y, x)
    px = r * jnp.sin(theta) * jnp.cos(phi)
    py = r * jnp.sin(theta) * jnp.sin(phi)
    pz = r * jnp.cos(theta)
    pn2 = jnp.sum(jnp.stack([px, py, pz], axis=-1) ** 2, axis=-1)
    KP = 8
    pp = jnp.concatenate(
        [px.reshape(B, 1, N), py.reshape(B, 1, N), pz.reshape(B, 1, N),
         pn2.reshape(B, 1, N),
         jnp.zeros((B, KP - 4, N), jnp.float32)], axis=1)
    np_idx = near_idx[0, 0]
    thr = jnp.sqrt(jnp.sum((vertex[0] - vertex[np_idx]) ** 2))
    t2 = thr * thr
    for _ in range(4):
        up = jnp.nextafter(t2, jnp.float32(jnp.inf))
        t2 = jnp.where(jnp.sqrt(up) <= thr, up, t2)
    for _ in range(4):
        dn = jnp.nextafter(t2, jnp.float32(-jnp.inf))
        t2 = jnp.where(jnp.sqrt(t2) <= thr, t2, dn)

    idx_flat = _first_hit_indices(pp, vertex, vn2, t2, B, N, NV, KP)

    D = 128
    NVp = NV + 128
    table = jnp.pad(jnp.transpose(projected, (0, 2, 1)),
                    ((0, 0), (0, NVp - NV), (0, D - C0)))
    feat = _sc_gather(table.reshape(B * NVp, D), idx_flat,
                      B, N, NVp, D).reshape(B, N, D)

    return _mlp(original, feat, W1, b1, g1, be1, W2, b2, g2, be2,
                B, C19, C0)


# speedup vs baseline: 1.1416x; 1.1416x over previous
import functools

import jax
import jax.numpy as jnp
from jax import lax
from jax.experimental import pallas as pl
from jax.experimental.pallas import tpu as pltpu
from jax.experimental.pallas import tpu_sc as plsc


def _first_hit_indices(pxyzT, pn2r, vertex, vn2, t2, B, N, NV, KP):
    NB = 256
    NG = N // NB

    vert_p = jnp.concatenate(
        [vertex, jnp.zeros((NV, KP - 3), jnp.float32)], axis=1)
    vn2c = vn2[:, None]
    t2s = t2.reshape(1, 1)

    def body(t2_ref, pxyz_ref, pn2_ref, vert_ref, vn2_ref, idx_ref):
        cross = lax.dot_general(
            vert_ref[...], pxyz_ref[0],
            (((1,), (0,)), ((), ())),
            preferred_element_type=jnp.float32)
        d2 = (pn2_ref[0] + vn2_ref[...]) - 2.0 * cross
        mask = d2 <= t2_ref[0, 0]
        vio = lax.broadcasted_iota(jnp.int32, (NV, NB), 0)
        cand = jnp.where(mask, vio, NV)
        first = jnp.min(cand, axis=0, keepdims=True)
        idx_ref[0] = first

    idx3 = pl.pallas_call(
        body,
        grid=(B, NG),
        in_specs=[
            pl.BlockSpec(memory_space=pltpu.SMEM),
            pl.BlockSpec((1, KP, NB), lambda b, g: (b, 0, g)),
            pl.BlockSpec((1, 1, NB), lambda b, g: (b, 0, g)),
            pl.BlockSpec((NV, KP), lambda b, g: (0, 0)),
            pl.BlockSpec((NV, 1), lambda b, g: (0, 0)),
        ],
        out_specs=pl.BlockSpec((1, 1, NB), lambda b, g: (b * NG + g, 0, 0)),
        out_shape=jax.ShapeDtypeStruct((B * NG, 1, NB), jnp.int32),
    )(t2s, pxyzT, pn2r, vert_p, vn2c)
    return idx3.reshape(B * N)


def _sc_gather(table, idx_local, B, N, NVp, D):
    info = plsc.get_sparse_core_info()
    NC = info.num_cores
    NS = info.num_subcores
    PPT = N // NS
    RPT = NVp // NS
    mesh = plsc.VectorSubcoreMesh(core_axis_name="c", subcore_axis_name="s")

    @functools.partial(
        pl.kernel, mesh=mesh,
        out_type=jax.ShapeDtypeStruct((B * N, D), jnp.float32),
        scratch_types=[
            pltpu.VMEM((PPT,), jnp.int32),
            pltpu.VMEM((PPT, D), jnp.float32),
            pltpu.VMEM_SHARED((NVp, D), jnp.float32),
            pltpu.SemaphoreType.DMA,
        ],
    )
    def gather_k(tbl_hbm, idx_hbm, out_hbm, idx_v, rows_v, shared, sem):
        c = lax.axis_index("c")
        s = lax.axis_index("s")
        pltpu.sync_copy(tbl_hbm.at[pl.ds(c * NVp + s * RPT, RPT)],
                        shared.at[pl.ds(s * RPT, RPT)])
        base = c * N + s * PPT
        pltpu.sync_copy(idx_hbm.at[pl.ds(base, PPT)], idx_v)
        plsc.subcore_barrier()
        pltpu.async_copy(shared.at[idx_v], rows_v, sem).wait()
        pltpu.sync_copy(rows_v, out_hbm.at[pl.ds(base, PPT)])

    return gather_k(table, idx_local)


def _mlp(original, feat, W1, b1, g1, be1, W2, b2, g2, be2, B, C19, C0):
    Ltot = float(B * (C19 + C0))
    col = lambda v: v.reshape(-1, 1)
    eps = 1e-5

    def body(orig_ref, feat_ref, W1_ref, W2_ref, b1_ref, g1_ref,
             be1_ref, b2_ref, g2_ref, be2_ref, out_ref):
        W1_ = W1_ref[...]
        cdims = (((1,), (0,)), ((), ()))
        hA = [lax.dot_general(W1_, orig_ref[b], (((1,), (1,)), ((), ())),
                              preferred_element_type=jnp.float32)
              + b1_ref[...] for b in range(B)]
        hF = [lax.dot_general(W1_, feat_ref[b][:, :C0], cdims,
                              preferred_element_type=jnp.float32)
              + b1_ref[...] for b in range(B)]

        def bn_relu(parts, g_ref, be_ref):
            s = sum(jnp.sum(h, axis=1, keepdims=True) for h in parts)
            mean = s / Ltot
            q = sum(jnp.sum((h - mean) ** 2, axis=1, keepdims=True)
                    for h in parts)
            scale = g_ref[...] / jnp.sqrt(q / Ltot + eps)
            return [jnp.maximum((h - mean) * scale + be_ref[...], 0.0)
                    for h in parts]

        y1 = bn_relu(hA + hF, g1_ref, be1_ref)
        W2_ = W2_ref[...]
        z = [lax.dot_general(W2_, y, cdims,
                             preferred_element_type=jnp.float32)
             + b2_ref[...] for y in y1]
        y2 = bn_relu(z, g2_ref, be2_ref)
        for b in range(B):
            out_ref[b, :, :C19] = y2[b]
            out_ref[b, :, C19:] = y2[B + b]

    M = W1.shape[0]
    return pl.pallas_call(
        body,
        out_shape=jax.ShapeDtypeStruct((B, M, C19 + C0), jnp.float32),
    )(original, feat, W1, W2, col(b1), col(g1), col(be1),
      col(b2), col(g2), col(be2))


def kernel(original, projected, vertex, near_idx,
           W1, b1, g1, be1, W2, b2, g2, be2):
    B, C19, N = original.shape
    NV = vertex.shape[0]
    C0 = projected.shape[1]

    r = jnp.sqrt(jnp.sum(vertex[0] ** 2))
    vn2 = jnp.sum(vertex ** 2, axis=-1)
    xyzr = original[:, :3, :].reshape(B * 3, N // 128, 128)
    x, y, z = xyzr[0::3], xyzr[1::3], xyzr[2::3]
    r_pt = jnp.sqrt(jnp.sum(jnp.stack([x, y, z], axis=-1) ** 2, axis=-1))
    theta = jnp.arccos(jnp.clip(z / jnp.maximum(r_pt, 1e-12), -1.0, 1.0))
    phi = jnp.arctan2(y, x)
    px = r * jnp.sin(theta) * jnp.cos(phi)
    py = r * jnp.sin(theta) * jnp.sin(phi)
    pz = r * jnp.cos(theta)
    pn2 = jnp.sum(jnp.stack([px, py, pz], axis=-1) ** 2, axis=-1)
    KP = 8
    pxyzT = jnp.concatenate(
        [px.reshape(B, 1, N), py.reshape(B, 1, N), pz.reshape(B, 1, N),
         jnp.zeros((B, KP - 3, N), jnp.float32)], axis=1)
    pn2r = pn2.reshape(B, 1, N)
    np_idx = near_idx[0, 0]
    thr = jnp.sqrt(jnp.sum((vertex[0] - vertex[np_idx]) ** 2))
    t2 = thr * thr
    for _ in range(4):
        up = jnp.nextafter(t2, jnp.float32(jnp.inf))
        t2 = jnp.where(jnp.sqrt(up) <= thr, up, t2)
    for _ in range(4):
        dn = jnp.nextafter(t2, jnp.float32(-jnp.inf))
        t2 = jnp.where(jnp.sqrt(t2) <= thr, t2, dn)

    idx_flat = _first_hit_indices(pxyzT, pn2r, vertex, vn2, t2, B, N, NV, KP)

    D = 128
    NVp = NV + 128
    table = jnp.pad(jnp.transpose(projected, (0, 2, 1)),
                    ((0, 0), (0, NVp - NV), (0, D - C0)))
    feat = _sc_gather(table.reshape(B * NVp, D), idx_flat,
                      B, N, NVp, D).reshape(B, N, D)

    return _mlp(original, feat, W1, b1, g1, be1, W2, b2, g2, be2,
                B, C19, C0)


# speedup vs baseline: 1.2104x; 1.0603x over previous
import functools

import jax
import jax.numpy as jnp
from jax import lax
from jax.experimental import pallas as pl
from jax.experimental.pallas import tpu as pltpu
from jax.experimental.pallas import tpu_sc as plsc


def _first_hit_indices(pxyzT, pn2r, vertex, vn2, t2, B, N, NV, KP):
    NB = 512
    NG = N // NB

    vert_p = jnp.concatenate(
        [vertex, jnp.zeros((NV, KP - 3), jnp.float32)], axis=1)
    vn2c = vn2[:, None]
    t2s = t2.reshape(1, 1)

    def body(t2_ref, pxyz_ref, pn2_ref, vert_ref, vn2_ref, idx_ref):
        cross = lax.dot_general(
            vert_ref[...], pxyz_ref[0],
            (((1,), (0,)), ((), ())),
            preferred_element_type=jnp.float32)
        d2 = (pn2_ref[0] + vn2_ref[...]) - 2.0 * cross
        mask = d2 <= t2_ref[0, 0]
        vio = lax.broadcasted_iota(jnp.int32, (NV, NB), 0)
        cand = jnp.where(mask, vio, NV)
        first = jnp.min(cand, axis=0, keepdims=True)
        idx_ref[0] = first

    idx3 = pl.pallas_call(
        body,
        grid=(B, NG),
        in_specs=[
            pl.BlockSpec(memory_space=pltpu.SMEM),
            pl.BlockSpec((1, KP, NB), lambda b, g: (b, 0, g)),
            pl.BlockSpec((1, 1, NB), lambda b, g: (b, 0, g)),
            pl.BlockSpec((NV, KP), lambda b, g: (0, 0)),
            pl.BlockSpec((NV, 1), lambda b, g: (0, 0)),
        ],
        out_specs=pl.BlockSpec((1, 1, NB), lambda b, g: (b * NG + g, 0, 0)),
        out_shape=jax.ShapeDtypeStruct((B * NG, 1, NB), jnp.int32),
    )(t2s, pxyzT, pn2r, vert_p, vn2c)
    return idx3.reshape(B * N)


def _sc_gather(table, idx_local, B, N, NVp, D):
    info = plsc.get_sparse_core_info()
    NC = info.num_cores
    NS = info.num_subcores
    PPT = N // NS
    RPT = NVp // NS
    mesh = plsc.VectorSubcoreMesh(core_axis_name="c", subcore_axis_name="s")

    @functools.partial(
        pl.kernel, mesh=mesh,
        out_type=jax.ShapeDtypeStruct((B * N, D), jnp.float32),
        scratch_types=[
            pltpu.VMEM((PPT,), jnp.int32),
            pltpu.VMEM((PPT, D), jnp.float32),
            pltpu.VMEM_SHARED((NVp, D), jnp.float32),
            pltpu.SemaphoreType.DMA,
        ],
    )
    def gather_k(tbl_hbm, idx_hbm, out_hbm, idx_v, rows_v, shared, sem):
        c = lax.axis_index("c")
        s = lax.axis_index("s")
        pltpu.sync_copy(tbl_hbm.at[pl.ds(c * NVp + s * RPT, RPT)],
                        shared.at[pl.ds(s * RPT, RPT)])
        base = c * N + s * PPT
        pltpu.sync_copy(idx_hbm.at[pl.ds(base, PPT)], idx_v)
        plsc.subcore_barrier()
        pltpu.async_copy(shared.at[idx_v], rows_v, sem).wait()
        pltpu.sync_copy(rows_v, out_hbm.at[pl.ds(base, PPT)])

    return gather_k(table, idx_local)


def _mlp(original, feat, W1, b1, g1, be1, W2, b2, g2, be2, B, C19, C0):
    Ltot = float(B * (C19 + C0))
    col = lambda v: v.reshape(-1, 1)
    eps = 1e-5

    def body(orig_ref, feat_ref, W1_ref, W2_ref, b1_ref, g1_ref,
             be1_ref, b2_ref, g2_ref, be2_ref, out_ref):
        W1_ = W1_ref[...]
        cdims = (((1,), (0,)), ((), ()))
        hA = [lax.dot_general(W1_, orig_ref[b], (((1,), (1,)), ((), ())),
                              preferred_element_type=jnp.float32)
              + b1_ref[...] for b in range(B)]
        hF = [lax.dot_general(W1_, feat_ref[b][:, :C0], cdims,
                              preferred_element_type=jnp.float32)
              + b1_ref[...] for b in range(B)]

        def bn_relu(parts, g_ref, be_ref):
            s = sum(jnp.sum(h, axis=1, keepdims=True) for h in parts)
            mean = s / Ltot
            q = sum(jnp.sum((h - mean) ** 2, axis=1, keepdims=True)
                    for h in parts)
            scale = g_ref[...] / jnp.sqrt(q / Ltot + eps)
            return [jnp.maximum((h - mean) * scale + be_ref[...], 0.0)
                    for h in parts]

        y1 = bn_relu(hA + hF, g1_ref, be1_ref)
        W2_ = W2_ref[...]
        z = [lax.dot_general(W2_, y, cdims,
                             preferred_element_type=jnp.float32)
             + b2_ref[...] for y in y1]
        y2 = bn_relu(z, g2_ref, be2_ref)
        for b in range(B):
            out_ref[b, :, :C19] = y2[b]
            out_ref[b, :, C19:] = y2[B + b]

    M = W1.shape[0]
    return pl.pallas_call(
        body,
        out_shape=jax.ShapeDtypeStruct((B, M, C19 + C0), jnp.float32),
    )(original, feat, W1, W2, col(b1), col(g1), col(be1),
      col(b2), col(g2), col(be2))


def kernel(original, projected, vertex, near_idx,
           W1, b1, g1, be1, W2, b2, g2, be2):
    B, C19, N = original.shape
    NV = vertex.shape[0]
    C0 = projected.shape[1]

    r = jnp.sqrt(jnp.sum(vertex[0] ** 2))
    vn2 = jnp.sum(vertex ** 2, axis=-1)
    xyzr = original[:, :3, :].reshape(B * 3, N // 128, 128)
    x, y, z = xyzr[0::3], xyzr[1::3], xyzr[2::3]
    r_pt = jnp.sqrt(jnp.sum(jnp.stack([x, y, z], axis=-1) ** 2, axis=-1))
    theta = jnp.arccos(jnp.clip(z / jnp.maximum(r_pt, 1e-12), -1.0, 1.0))
    phi = jnp.arctan2(y, x)
    px = r * jnp.sin(theta) * jnp.cos(phi)
    py = r * jnp.sin(theta) * jnp.sin(phi)
    pz = r * jnp.cos(theta)
    pn2 = jnp.sum(jnp.stack([px, py, pz], axis=-1) ** 2, axis=-1)
    KP = 8
    pxyzT = jnp.concatenate(
        [px.reshape(B, 1, N), py.reshape(B, 1, N), pz.reshape(B, 1, N),
         jnp.zeros((B, KP - 3, N), jnp.float32)], axis=1)
    pn2r = pn2.reshape(B, 1, N)
    np_idx = near_idx[0, 0]
    thr = jnp.sqrt(jnp.sum((vertex[0] - vertex[np_idx]) ** 2))
    t2 = thr * thr
    for _ in range(4):
        up = jnp.nextafter(t2, jnp.float32(jnp.inf))
        t2 = jnp.where(jnp.sqrt(up) <= thr, up, t2)
    for _ in range(4):
        dn = jnp.nextafter(t2, jnp.float32(-jnp.inf))
        t2 = jnp.where(jnp.sqrt(t2) <= thr, t2, dn)

    idx_flat = _first_hit_indices(pxyzT, pn2r, vertex, vn2, t2, B, N, NV, KP)

    D = 128
    NVp = NV + 128
    table = jnp.pad(jnp.transpose(projected, (0, 2, 1)),
                    ((0, 0), (0, NVp - NV), (0, D - C0)))
    feat = _sc_gather(table.reshape(B * NVp, D), idx_flat,
                      B, N, NVp, D).reshape(B, N, D)

    return _mlp(original, feat, W1, b1, g1, be1, W2, b2, g2, be2,
                B, C19, C0)


# speedup vs baseline: 1.2271x; 1.0138x over previous
import functools

import jax
import jax.numpy as jnp
from jax import lax
from jax.experimental import pallas as pl
from jax.experimental.pallas import tpu as pltpu
from jax.experimental.pallas import tpu_sc as plsc


def _first_hit_indices(pxyzT, pn2r, vertex, vn2, t2, B, N, NV, KP):
    NB = 1024
    NG = N // NB

    vert_p = jnp.concatenate(
        [vertex, jnp.zeros((NV, KP - 3), jnp.float32)], axis=1)
    vn2c = vn2[:, None]
    t2s = t2.reshape(1, 1)

    def body(t2_ref, pxyz_ref, pn2_ref, vert_ref, vn2_ref, idx_ref):
        cross = lax.dot_general(
            vert_ref[...], pxyz_ref[0],
            (((1,), (0,)), ((), ())),
            preferred_element_type=jnp.float32)
        d2 = (pn2_ref[0] + vn2_ref[...]) - 2.0 * cross
        mask = d2 <= t2_ref[0, 0]
        vio = lax.broadcasted_iota(jnp.int32, (NV, NB), 0)
        cand = jnp.where(mask, vio, NV)
        first = jnp.min(cand, axis=0, keepdims=True)
        idx_ref[0] = first

    idx3 = pl.pallas_call(
        body,
        grid=(B, NG),
        in_specs=[
            pl.BlockSpec(memory_space=pltpu.SMEM),
            pl.BlockSpec((1, KP, NB), lambda b, g: (b, 0, g)),
            pl.BlockSpec((1, 1, NB), lambda b, g: (b, 0, g)),
            pl.BlockSpec((NV, KP), lambda b, g: (0, 0)),
            pl.BlockSpec((NV, 1), lambda b, g: (0, 0)),
        ],
        out_specs=pl.BlockSpec((1, 1, NB), lambda b, g: (b * NG + g, 0, 0)),
        out_shape=jax.ShapeDtypeStruct((B * NG, 1, NB), jnp.int32),
    )(t2s, pxyzT, pn2r, vert_p, vn2c)
    return idx3.reshape(B * N)


def _sc_gather(table, idx_local, B, N, NVp, D):
    info = plsc.get_sparse_core_info()
    NC = info.num_cores
    NS = info.num_subcores
    PPT = N // NS
    RPT = NVp // NS
    mesh = plsc.VectorSubcoreMesh(core_axis_name="c", subcore_axis_name="s")

    @functools.partial(
        pl.kernel, mesh=mesh,
        out_type=jax.ShapeDtypeStruct((B * N, D), jnp.float32),
        scratch_types=[
            pltpu.VMEM((PPT,), jnp.int32),
            pltpu.VMEM((PPT, D), jnp.float32),
            pltpu.VMEM_SHARED((NVp, D), jnp.float32),
            pltpu.SemaphoreType.DMA,
        ],
    )
    def gather_k(tbl_hbm, idx_hbm, out_hbm, idx_v, rows_v, shared, sem):
        c = lax.axis_index("c")
        s = lax.axis_index("s")
        pltpu.sync_copy(tbl_hbm.at[pl.ds(c * NVp + s * RPT, RPT)],
                        shared.at[pl.ds(s * RPT, RPT)])
        base = c * N + s * PPT
        pltpu.sync_copy(idx_hbm.at[pl.ds(base, PPT)], idx_v)
        plsc.subcore_barrier()
        pltpu.async_copy(shared.at[idx_v], rows_v, sem).wait()
        pltpu.sync_copy(rows_v, out_hbm.at[pl.ds(base, PPT)])

    return gather_k(table, idx_local)


def _mlp(original, feat, W1, b1, g1, be1, W2, b2, g2, be2, B, C19, C0):
    Ltot = float(B * (C19 + C0))
    col = lambda v: v.reshape(-1, 1)
    eps = 1e-5

    def body(orig_ref, feat_ref, W1_ref, W2_ref, b1_ref, g1_ref,
             be1_ref, b2_ref, g2_ref, be2_ref, out_ref):
        W1_ = W1_ref[...]
        cdims = (((1,), (0,)), ((), ()))
        hA = [lax.dot_general(W1_, orig_ref[b], (((1,), (1,)), ((), ())),
                              preferred_element_type=jnp.float32)
              + b1_ref[...] for b in range(B)]
        hF = [lax.dot_general(W1_, feat_ref[b][:, :C0], cdims,
                              preferred_element_type=jnp.float32)
              + b1_ref[...] for b in range(B)]

        def bn_relu(parts, g_ref, be_ref):
            s = sum(jnp.sum(h, axis=1, keepdims=True) for h in parts)
            mean = s / Ltot
            q = sum(jnp.sum((h - mean) ** 2, axis=1, keepdims=True)
                    for h in parts)
            scale = g_ref[...] / jnp.sqrt(q / Ltot + eps)
            return [jnp.maximum((h - mean) * scale + be_ref[...], 0.0)
                    for h in parts]

        y1 = bn_relu(hA + hF, g1_ref, be1_ref)
        W2_ = W2_ref[...]
        z = [lax.dot_general(W2_, y, cdims,
                             preferred_element_type=jnp.float32)
             + b2_ref[...] for y in y1]
        y2 = bn_relu(z, g2_ref, be2_ref)
        for b in range(B):
            out_ref[b, :, :C19] = y2[b]
            out_ref[b, :, C19:] = y2[B + b]

    M = W1.shape[0]
    return pl.pallas_call(
        body,
        out_shape=jax.ShapeDtypeStruct((B, M, C19 + C0), jnp.float32),
    )(original, feat, W1, W2, col(b1), col(g1), col(be1),
      col(b2), col(g2), col(be2))


def kernel(original, projected, vertex, near_idx,
           W1, b1, g1, be1, W2, b2, g2, be2):
    B, C19, N = original.shape
    NV = vertex.shape[0]
    C0 = projected.shape[1]

    r = jnp.sqrt(jnp.sum(vertex[0] ** 2))
    vn2 = jnp.sum(vertex ** 2, axis=-1)
    xyzr = original[:, :3, :].reshape(B * 3, N // 128, 128)
    x, y, z = xyzr[0::3], xyzr[1::3], xyzr[2::3]
    r_pt = jnp.sqrt(jnp.sum(jnp.stack([x, y, z], axis=-1) ** 2, axis=-1))
    theta = jnp.arccos(jnp.clip(z / jnp.maximum(r_pt, 1e-12), -1.0, 1.0))
    phi = jnp.arctan2(y, x)
    px = r * jnp.sin(theta) * jnp.cos(phi)
    py = r * jnp.sin(theta) * jnp.sin(phi)
    pz = r * jnp.cos(theta)
    pn2 = jnp.sum(jnp.stack([px, py, pz], axis=-1) ** 2, axis=-1)
    KP = 8
    pxyzT = jnp.concatenate(
        [px.reshape(B, 1, N), py.reshape(B, 1, N), pz.reshape(B, 1, N),
         jnp.zeros((B, KP - 3, N), jnp.float32)], axis=1)
    pn2r = pn2.reshape(B, 1, N)
    np_idx = near_idx[0, 0]
    thr = jnp.sqrt(jnp.sum((vertex[0] - vertex[np_idx]) ** 2))
    t2 = thr * thr
    for _ in range(4):
        up = jnp.nextafter(t2, jnp.float32(jnp.inf))
        t2 = jnp.where(jnp.sqrt(up) <= thr, up, t2)
    for _ in range(4):
        dn = jnp.nextafter(t2, jnp.float32(-jnp.inf))
        t2 = jnp.where(jnp.sqrt(t2) <= thr, t2, dn)

    idx_flat = _first_hit_indices(pxyzT, pn2r, vertex, vn2, t2, B, N, NV, KP)

    D = 128
    NVp = NV + 128
    table = jnp.pad(jnp.transpose(projected, (0, 2, 1)),
                    ((0, 0), (0, NVp - NV), (0, D - C0)))
    feat = _sc_gather(table.reshape(B * NVp, D), idx_flat,
                      B, N, NVp, D).reshape(B, N, D)

    return _mlp(original, feat, W1, b1, g1, be1, W2, b2, g2, be2,
                B, C19, C0)
